# baseline (speedup 1.0000x reference)
import functools

import jax
import jax.numpy as jnp
from jax import lax
from jax.experimental import pallas as pl
from jax.experimental.pallas import tpu as pltpu
from jax.experimental.pallas import tpu_sc as plsc

_B = 16384
_EMB = 32
_H1 = 64
_NROWS = 1000000
_NC = 2
_NS = 16
_NW = _NC * _NS
_BPW = _B // _NW
_L = 16
_BANK = 8
_LANES = 128

_MLP_BLK = 2048


def _gather_body(uids, iids, utab, itab, uout, iout, uids_v, iids_v,
                 ring_u, ring_i, ubuf, ibuf, usem, isem):
    wid = lax.axis_index("s") * _NC + lax.axis_index("c")
    base = pl.multiple_of(wid * _BPW, _BPW)
    pltpu.sync_copy(uids.at[pl.ds(base, _BPW)], uids_v.at[pl.ds(0, _BPW)])
    pltpu.sync_copy(iids.at[pl.ds(base, _BPW)], iids_v.at[pl.ds(0, _BPW)])

    lanes = lax.iota(jnp.int32, _L)

    def fire(tab, rt16, ring, sem):
        copies = []
        for t in range(_BANK):
            off = pl.multiple_of(rt16[t] * 8, 8)
            copies.append(
                pltpu.async_copy(tab.at[pl.ds(off, 8), :], ring.at[t], sem))
        return copies

    def extract(su16, ring, obuf, g):
        for t in range(_BANK):
            col = jnp.full((_L,), g * _BANK + t, jnp.int32)
            suv = jnp.full((_L,), su16[t], jnp.int32)
            tv = jnp.full((_L,), t, jnp.int32)
            lo = plsc.load_gather(ring, [tv, suv, lanes])
            hi = plsc.load_gather(ring, [tv, suv, lanes + _L])
            plsc.store_scatter(obuf, [lanes, col], lo)
            plsc.store_scatter(obuf, [lanes + _L, col], hi)

    def group(g, _):
        u16 = uids_v[pl.ds(g * _BANK, _L)]
        i16 = iids_v[pl.ds(g * _BANK, _L)]
        urt = lax.shift_right_logical(u16, 3)
        usu = lax.bitwise_and(u16, 7)
        irt = lax.shift_right_logical(i16, 3)
        isu = lax.bitwise_and(i16, 7)
        ucp = fire(utab, urt, ring_u, usem)
        icp = fire(itab, irt, ring_i, isem)
        for cp in ucp:
            cp.wait()
        extract(usu, ring_u, ubuf, g)
        for cp in icp:
            cp.wait()
        extract(isu, ring_i, ibuf, g)
        return 0

    lax.fori_loop(0, _BPW // _BANK, group, 0)
    pltpu.sync_copy(ubuf, uout.at[:, pl.ds(base, _BPW)])
    pltpu.sync_copy(ibuf, iout.at[:, pl.ds(base, _BPW)])


@jax.jit
def _gather(uids, iids, utab, itab):
    mesh = plsc.VectorSubcoreMesh(core_axis_name="c", subcore_axis_name="s")
    fn = functools.partial(
        pl.kernel,
        mesh=mesh,
        out_type=(
            jax.ShapeDtypeStruct((_EMB, _B), jnp.float32),
            jax.ShapeDtypeStruct((_EMB, _B), jnp.float32),
        ),
        scratch_types=[
            pltpu.VMEM((_BPW + _L,), jnp.int32),
            pltpu.VMEM((_BPW + _L,), jnp.int32),
            pltpu.VMEM((_BANK, 8, _EMB), jnp.float32),
            pltpu.VMEM((_BANK, 8, _EMB), jnp.float32),
            pltpu.VMEM((_EMB, _BPW), jnp.float32),
            pltpu.VMEM((_EMB, _BPW), jnp.float32),
            pltpu.SemaphoreType.DMA,
            pltpu.SemaphoreType.DMA,
        ],
        compiler_params=pltpu.CompilerParams(needs_layout_passes=False),
    )(_gather_body)
    return fn(uids, iids, utab, itab)


def _mlp_body(u_ref, i_ref, w1a_ref, w1b_ref, b1_ref, w2_ref, b2_ref,
              w3_ref, b3_ref, o_ref):
    u = u_ref[...]
    v = i_ref[...]
    h = jnp.dot(w1a_ref[...], u, preferred_element_type=jnp.float32)
    h = h + jnp.dot(w1b_ref[...], v, preferred_element_type=jnp.float32)
    h = jnp.maximum(h + b1_ref[...], 0.0)
    h = jnp.dot(w2_ref[...], h, preferred_element_type=jnp.float32)
    h = jnp.maximum(h + b2_ref[...], 0.0)
    logit = jnp.sum(h * w3_ref[...], axis=0) + b3_ref[0]
    o_ref[...] = 1.0 / (1.0 + jnp.exp(-logit))


@jax.jit
def _mlp(uembT, iembT, w1aT, w1bT, b1c, w2T, b2c, w3c, b3):
    grid = (_B // _MLP_BLK,)
    return pl.pallas_call(
        _mlp_body,
        grid=grid,
        in_specs=[
            pl.BlockSpec((_EMB, _MLP_BLK), lambda i: (0, i)),
            pl.BlockSpec((_EMB, _MLP_BLK), lambda i: (0, i)),
            pl.BlockSpec((_H1, _EMB), lambda i: (0, 0)),
            pl.BlockSpec((_H1, _EMB), lambda i: (0, 0)),
            pl.BlockSpec((_H1, 1), lambda i: (0, 0)),
            pl.BlockSpec((_EMB, _H1), lambda i: (0, 0)),
            pl.BlockSpec((_EMB, 1), lambda i: (0, 0)),
            pl.BlockSpec((_EMB, 1), lambda i: (0, 0)),
            pl.BlockSpec(memory_space=pltpu.SMEM),
        ],
        out_specs=pl.BlockSpec((_MLP_BLK,), lambda i: (i,)),
        out_shape=jax.ShapeDtypeStruct((_B,), jnp.float32),
    )(uembT, iembT, w1aT, w1bT, b1c, w2T, b2c, w3c, b3)


def kernel(user_ids, item_ids, user_table, item_table, W1, b1, W2, b2, W3,
           b3):
    uids = user_ids.astype(jnp.int32)
    iids = item_ids.astype(jnp.int32)
    uembT, iembT = _gather(uids, iids, user_table, item_table)
    return _mlp(
        uembT, iembT,
        W1[:_EMB].T, W1[_EMB:].T,
        b1.reshape(_H1, 1),
        W2.T,
        b2.reshape(_EMB, 1),
        W3.reshape(_EMB, 1),
        b3.reshape(1),
    )


# speedup vs baseline: 1.0694x; 1.0694x over previous
import functools

import jax
import jax.numpy as jnp
from jax import lax
from jax.experimental import pallas as pl
from jax.experimental.pallas import tpu as pltpu
from jax.experimental.pallas import tpu_sc as plsc

_B = 16384
_EMB = 32
_H1 = 64
_NROWS = 1000000
_NC = 2
_NS = 16
_NW = _NC * _NS
_BPW = _B // _NW
_L = 16
_BANK = 16

_MLP_BLK = 2048


def _gather_body(uids, iids, utab, itab, uout, iout, uids_v, iids_v,
                 ring_u, ring_i, ubuf, ibuf, usem, isem):
    wid = lax.axis_index("s") * _NC + lax.axis_index("c")
    base = pl.multiple_of(wid * _BPW, _BPW)
    pltpu.sync_copy(uids.at[pl.ds(base, _BPW)], uids_v.at[pl.ds(0, _BPW)])
    pltpu.sync_copy(iids.at[pl.ds(base, _BPW)], iids_v.at[pl.ds(0, _BPW)])

    tvec = lax.iota(jnp.int32, _L)

    def fire(tab, idv, ring, sem):
        copies = []
        for t in range(_BANK):
            copies.append(
                pltpu.async_copy(tab.at[pl.ds(idv[t], 1), :],
                                 ring.at[pl.ds(t, 1), :], sem))
        return copies

    def extract(ring, obuf, g):
        col = g * _L
        for d in range(_EMB):
            dv = jnp.full((_L,), d, jnp.int32)
            vals = plsc.load_gather(ring, [tvec, dv])
            obuf[d, pl.ds(col, _L)] = vals

    def group(g, _):
        u16 = uids_v[pl.ds(g * _BANK, _L)]
        i16 = iids_v[pl.ds(g * _BANK, _L)]
        ucp = fire(utab, u16, ring_u, usem)
        icp = fire(itab, i16, ring_i, isem)
        for cp in ucp:
            cp.wait()
        extract(ring_u, ubuf, g)
        for cp in icp:
            cp.wait()
        extract(ring_i, ibuf, g)
        return 0

    lax.fori_loop(0, _BPW // _BANK, group, 0)
    pltpu.sync_copy(ubuf, uout.at[:, pl.ds(base, _BPW)])
    pltpu.sync_copy(ibuf, iout.at[:, pl.ds(base, _BPW)])


@jax.jit
def _gather(uids, iids, utab, itab):
    mesh = plsc.VectorSubcoreMesh(core_axis_name="c", subcore_axis_name="s")
    fn = functools.partial(
        pl.kernel,
        mesh=mesh,
        out_type=(
            jax.ShapeDtypeStruct((_EMB, _B), jnp.float32),
            jax.ShapeDtypeStruct((_EMB, _B), jnp.float32),
        ),
        scratch_types=[
            pltpu.VMEM((_BPW + _L,), jnp.int32),
            pltpu.VMEM((_BPW + _L,), jnp.int32),
            pltpu.VMEM((_BANK, _EMB), jnp.float32),
            pltpu.VMEM((_BANK, _EMB), jnp.float32),
            pltpu.VMEM((_EMB, _BPW), jnp.float32),
            pltpu.VMEM((_EMB, _BPW), jnp.float32),
            pltpu.SemaphoreType.DMA,
            pltpu.SemaphoreType.DMA,
        ],
        compiler_params=pltpu.CompilerParams(needs_layout_passes=False),
    )(_gather_body)
    return fn(uids, iids, utab, itab)


def _mlp_body(u_ref, i_ref, w1a_ref, w1b_ref, b1_ref, w2_ref, b2_ref,
              w3_ref, b3_ref, o_ref):
    u = u_ref[...]
    v = i_ref[...]
    h = jnp.dot(w1a_ref[...], u, preferred_element_type=jnp.float32)
    h = h + jnp.dot(w1b_ref[...], v, preferred_element_type=jnp.float32)
    h = jnp.maximum(h + b1_ref[...], 0.0)
    h = jnp.dot(w2_ref[...], h, preferred_element_type=jnp.float32)
    h = jnp.maximum(h + b2_ref[...], 0.0)
    logit = jnp.sum(h * w3_ref[...], axis=0) + b3_ref[0]
    o_ref[...] = 1.0 / (1.0 + jnp.exp(-logit))


@jax.jit
def _mlp(uembT, iembT, w1aT, w1bT, b1c, w2T, b2c, w3c, b3):
    grid = (_B // _MLP_BLK,)
    return pl.pallas_call(
        _mlp_body,
        grid=grid,
        in_specs=[
            pl.BlockSpec((_EMB, _MLP_BLK), lambda i: (0, i)),
            pl.BlockSpec((_EMB, _MLP_BLK), lambda i: (0, i)),
            pl.BlockSpec((_H1, _EMB), lambda i: (0, 0)),
            pl.BlockSpec((_H1, _EMB), lambda i: (0, 0)),
            pl.BlockSpec((_H1, 1), lambda i: (0, 0)),
            pl.BlockSpec((_EMB, _H1), lambda i: (0, 0)),
            pl.BlockSpec((_EMB, 1), lambda i: (0, 0)),
            pl.BlockSpec((_EMB, 1), lambda i: (0, 0)),
            pl.BlockSpec(memory_space=pltpu.SMEM),
        ],
        out_specs=pl.BlockSpec((_MLP_BLK,), lambda i: (i,)),
        out_shape=jax.ShapeDtypeStruct((_B,), jnp.float32),
    )(uembT, iembT, w1aT, w1bT, b1c, w2T, b2c, w3c, b3)


def kernel(user_ids, item_ids, user_table, item_table, W1, b1, W2, b2, W3,
           b3):
    uids = user_ids.astype(jnp.int32)
    iids = item_ids.astype(jnp.int32)
    uembT, iembT = _gather(uids, iids, user_table, item_table)
    return _mlp(
        uembT, iembT,
        W1[:_EMB].T, W1[_EMB:].T,
        b1.reshape(_H1, 1),
        W2.T,
        b2.reshape(_EMB, 1),
        W3.reshape(_EMB, 1),
        b3.reshape(1),
    )


# speedup vs baseline: 2.8961x; 2.7081x over previous
import functools

import jax
import jax.numpy as jnp
from jax import lax
from jax.experimental import pallas as pl
from jax.experimental.pallas import tpu as pltpu
from jax.experimental.pallas import tpu_sc as plsc

_B = 16384
_EMB = 32
_H1 = 64
_NROWS = 1000000
_NC = 2
_NS = 16
_NW = _NC * _NS
_BPW = _B // _NW
_L = 16
_HB = 4
_LANES = 128

_MLP_BLK = 2048


def _gather_body(uids, iids, utab, itab, uout, iout, uids_v, iids_v,
                 ring_u, ring_i, ubuf, ibuf, usem, isem):
    wid = lax.axis_index("s") * _NC + lax.axis_index("c")
    base = pl.multiple_of(wid * _BPW, _BPW)
    pltpu.sync_copy(uids.at[pl.ds(base, _BPW)], uids_v.at[pl.ds(0, _BPW)])
    pltpu.sync_copy(iids.at[pl.ds(base, _BPW)], iids_v.at[pl.ds(0, _BPW)])

    lanes = lax.iota(jnp.int32, _L)

    def fire(r, h):
        u4 = uids_v[pl.ds(r * _HB, _L)]
        i4 = iids_v[pl.ds(r * _HB, _L)]
        ucp, icp = [], []
        for t in range(_HB):
            off = pl.multiple_of(
                lax.shift_right_logical(u4[t], 7) * _LANES, _LANES)
            ucp.append(pltpu.async_copy(utab.at[:, pl.ds(off, _LANES)],
                                        ring_u.at[h, t], usem))
        for t in range(_HB):
            off = pl.multiple_of(
                lax.shift_right_logical(i4[t], 7) * _LANES, _LANES)
            icp.append(pltpu.async_copy(itab.at[:, pl.ds(off, _LANES)],
                                        ring_i.at[h, t], isem))
        return ucp, icp

    def extract(r, h, su4, ring, obuf):
        hv = jnp.full((_L,), h, jnp.int32)
        for t in range(_HB):
            col = jnp.full((_L,), r * _HB + t, jnp.int32)
            suv = jnp.full((_L,), su4[t], jnp.int32)
            tv = jnp.full((_L,), t, jnp.int32)
            lo = plsc.load_gather(ring, [hv, tv, lanes, suv])
            hi = plsc.load_gather(ring, [hv, tv, lanes + _L, suv])
            plsc.store_scatter(obuf, [lanes, col], lo)
            plsc.store_scatter(obuf, [lanes + _L, col], hi)

    def drain(r, h, ucp, icp):
        u4 = uids_v[pl.ds(r * _HB, _L)]
        i4 = iids_v[pl.ds(r * _HB, _L)]
        usu = lax.bitwise_and(u4, _LANES - 1)
        isu = lax.bitwise_and(i4, _LANES - 1)
        for cp in ucp:
            cp.wait()
        extract(r, h, usu, ring_u, ubuf)
        for cp in icp:
            cp.wait()
        extract(r, h, isu, ring_i, ibuf)

    _NR = _BPW // _HB

    def pair(k, _):
        ucp, icp = fire(2 * k + 1, 1)
        drain(2 * k, 0, ucp, icp)
        ucp, icp = fire(2 * k + 2, 0)
        drain(2 * k + 1, 1, ucp, icp)
        return 0

    ucp0, icp0 = fire(0, 0)
    lax.fori_loop(0, _NR // 2 - 1, pair, 0)
    ucp, icp = fire(_NR - 1, 1)
    drain(_NR - 2, 0, ucp, icp)
    drain(_NR - 1, 1, ucp0, icp0)
    pltpu.sync_copy(ubuf, uout.at[:, pl.ds(base, _BPW)])
    pltpu.sync_copy(ibuf, iout.at[:, pl.ds(base, _BPW)])


@jax.jit
def _gather(uids, iids, utab, itab):
    mesh = plsc.VectorSubcoreMesh(core_axis_name="c", subcore_axis_name="s")
    fn = functools.partial(
        pl.kernel,
        mesh=mesh,
        out_type=(
            jax.ShapeDtypeStruct((_EMB, _B), jnp.float32),
            jax.ShapeDtypeStruct((_EMB, _B), jnp.float32),
        ),
        scratch_types=[
            pltpu.VMEM((_BPW + _L,), jnp.int32),
            pltpu.VMEM((_BPW + _L,), jnp.int32),
            pltpu.VMEM((2, _HB, _EMB, _LANES), jnp.float32),
            pltpu.VMEM((2, _HB, _EMB, _LANES), jnp.float32),
            pltpu.VMEM((_EMB, _BPW), jnp.float32),
            pltpu.VMEM((_EMB, _BPW), jnp.float32),
            pltpu.SemaphoreType.DMA,
            pltpu.SemaphoreType.DMA,
        ],
        compiler_params=pltpu.CompilerParams(needs_layout_passes=False),
    )(_gather_body)
    return fn(uids, iids, utab, itab)


def _mlp_body(u_ref, i_ref, w1a_ref, w1b_ref, b1_ref, w2_ref, b2_ref,
              w3_ref, b3_ref, o_ref):
    u = u_ref[...]
    v = i_ref[...]
    h = jnp.dot(w1a_ref[...], u, preferred_element_type=jnp.float32)
    h = h + jnp.dot(w1b_ref[...], v, preferred_element_type=jnp.float32)
    h = jnp.maximum(h + b1_ref[...], 0.0)
    h = jnp.dot(w2_ref[...], h, preferred_element_type=jnp.float32)
    h = jnp.maximum(h + b2_ref[...], 0.0)
    logit = jnp.sum(h * w3_ref[...], axis=0) + b3_ref[0]
    o_ref[...] = 1.0 / (1.0 + jnp.exp(-logit))


@jax.jit
def _mlp(uembT, iembT, w1aT, w1bT, b1c, w2T, b2c, w3c, b3):
    grid = (_B // _MLP_BLK,)
    return pl.pallas_call(
        _mlp_body,
        grid=grid,
        in_specs=[
            pl.BlockSpec((_EMB, _MLP_BLK), lambda i: (0, i)),
            pl.BlockSpec((_EMB, _MLP_BLK), lambda i: (0, i)),
            pl.BlockSpec((_H1, _EMB), lambda i: (0, 0)),
            pl.BlockSpec((_H1, _EMB), lambda i: (0, 0)),
            pl.BlockSpec((_H1, 1), lambda i: (0, 0)),
            pl.BlockSpec((_EMB, _H1), lambda i: (0, 0)),
            pl.BlockSpec((_EMB, 1), lambda i: (0, 0)),
            pl.BlockSpec((_EMB, 1), lambda i: (0, 0)),
            pl.BlockSpec(memory_space=pltpu.SMEM),
        ],
        out_specs=pl.BlockSpec((_MLP_BLK,), lambda i: (i,)),
        out_shape=jax.ShapeDtypeStruct((_B,), jnp.float32),
    )(uembT, iembT, w1aT, w1bT, b1c, w2T, b2c, w3c, b3)


def kernel(user_ids, item_ids, user_table, item_table, W1, b1, W2, b2, W3,
           b3):
    uids = user_ids.astype(jnp.int32)
    iids = item_ids.astype(jnp.int32)
    uembT, iembT = _gather(uids, iids, user_table.T, item_table.T)
    return _mlp(
        uembT, iembT,
        W1[:_EMB].T, W1[_EMB:].T,
        b1.reshape(_H1, 1),
        W2.T,
        b2.reshape(_EMB, 1),
        W3.reshape(_EMB, 1),
        b3.reshape(1),
    )
